# baseline (speedup 1.0000x reference)
import functools

import jax
import jax.numpy as jnp
from jax.experimental import pallas as pl
from jax.experimental.pallas import tpu as pltpu

K = 8
LANES = 128


def _row_imap(i, idx_ref, tgt_ref, *, k):
    return (idx_ref[i * K + k], 0, 0)


def _fused_body(idx_ref, tgt_ref, *refs):
    row_refs = refs[:K]
    emb_ref, loss_ref = refs[K], refs[K + 1]
    step = pl.program_id(0)

    @pl.when(step == 0)
    def _():
        loss_ref[0, 0] = 0.0

    _, sub, lan = row_refs[0].shape
    sub_iota = jax.lax.broadcasted_iota(jnp.int32, (sub, lan), 0)
    lane_iota = jax.lax.broadcasted_iota(jnp.int32, (sub, lan), 1)
    acc = jnp.float32(0.0)
    for k in range(K):
        row = row_refs[k][0]
        emb_ref[k] = row
        m = jnp.max(row)
        s = jnp.sum(jnp.exp(row - m))
        lse = m + jnp.log(s)
        tgt = tgt_ref[step * K + k]
        hit = (sub_iota == tgt // lan) & (lane_iota == tgt % lan)
        tval = jnp.sum(jnp.where(hit, row, 0.0))
        acc += lse - tval
    loss_ref[0, 0] += acc


def kernel(indices, targets, table):
    b, t = indices.shape
    n = b * t
    vocab, v = table.shape
    sub = v // LANES
    idx_flat = indices.reshape(n)
    tgt_flat = targets.reshape(n)
    table3 = table.reshape(vocab, sub, LANES)

    grid_spec = pltpu.PrefetchScalarGridSpec(
        num_scalar_prefetch=2,
        grid=(n // K,),
        in_specs=[
            pl.BlockSpec((1, sub, LANES), functools.partial(_row_imap, k=k))
            for k in range(K)
        ],
        out_specs=[
            pl.BlockSpec((K, sub, LANES), lambda i, idx_ref, tgt_ref: (i, 0, 0)),
            pl.BlockSpec(
                (1, 1),
                lambda i, idx_ref, tgt_ref: (0, 0),
                memory_space=pltpu.SMEM,
            ),
        ],
    )
    emb, loss_sum = pl.pallas_call(
        _fused_body,
        grid_spec=grid_spec,
        out_shape=[
            jax.ShapeDtypeStruct((n, sub, LANES), jnp.float32),
            jax.ShapeDtypeStruct((1, 1), jnp.float32),
        ],
    )(idx_flat, tgt_flat, *([table3] * K))
    embeddings = emb.reshape(b, t, v)
    loss = loss_sum[0, 0] / n
    return (embeddings, loss)


# speedup vs baseline: 2.0675x; 2.0675x over previous
import functools

import jax
import jax.numpy as jnp
from jax.experimental import pallas as pl
from jax.experimental.pallas import tpu as pltpu

K = 16
LANES = 128


def _row_imap(i, idx_ref, tgt_ref, *, k):
    return (idx_ref[i * K + k], 0, 0)


def _fused_body(idx_ref, tgt_ref, *refs):
    row_refs = refs[:K]
    emb_ref, loss_ref = refs[K], refs[K + 1]
    step = pl.program_id(0)

    @pl.when(step == 0)
    def _():
        loss_ref[0, 0] = 0.0

    _, sub, lan = row_refs[0].shape
    rows = jnp.concatenate([r[...] for r in row_refs], axis=0)
    emb_ref[...] = rows
    sub_iota = jax.lax.broadcasted_iota(jnp.int32, (K, sub, lan), 1)
    lane_iota = jax.lax.broadcasted_iota(jnp.int32, (K, sub, lan), 2)
    tgts = jnp.array([tgt_ref[step * K + k] for k in range(K)]).reshape(K, 1, 1)
    m = jnp.max(rows, axis=(1, 2), keepdims=True)
    s = jnp.sum(jnp.exp(rows - m), axis=(1, 2))
    lse = m.reshape(K) + jnp.log(s)
    hit = (sub_iota == tgts // lan) & (lane_iota == tgts % lan)
    tval = jnp.sum(jnp.where(hit, rows, 0.0), axis=(1, 2))
    loss_ref[0, 0] += jnp.sum(lse - tval)


def kernel(indices, targets, table):
    b, t = indices.shape
    n = b * t
    vocab, v = table.shape
    sub = v // LANES
    idx_flat = indices.reshape(n)
    tgt_flat = targets.reshape(n)
    table3 = table.reshape(vocab, sub, LANES)

    grid_spec = pltpu.PrefetchScalarGridSpec(
        num_scalar_prefetch=2,
        grid=(n // K,),
        in_specs=[
            pl.BlockSpec((1, sub, LANES), functools.partial(_row_imap, k=k))
            for k in range(K)
        ],
        out_specs=[
            pl.BlockSpec((K, sub, LANES), lambda i, idx_ref, tgt_ref: (i, 0, 0)),
            pl.BlockSpec(
                (1, 1),
                lambda i, idx_ref, tgt_ref: (0, 0),
                memory_space=pltpu.SMEM,
            ),
        ],
    )
    emb, loss_sum = pl.pallas_call(
        _fused_body,
        grid_spec=grid_spec,
        out_shape=[
            jax.ShapeDtypeStruct((n, sub, LANES), jnp.float32),
            jax.ShapeDtypeStruct((1, 1), jnp.float32),
        ],
    )(idx_flat, tgt_flat, *([table3] * K))
    embeddings = emb.reshape(b, t, v)
    loss = loss_sum[0, 0] / n
    return (embeddings, loss)


# speedup vs baseline: 2.3024x; 1.1136x over previous
import functools

import jax
import jax.numpy as jnp
from jax.experimental import pallas as pl
from jax.experimental.pallas import tpu as pltpu

K = 32
LANES = 128


def _row_imap(i, idx_ref, tgt_ref, *, k):
    return (idx_ref[i * K + k], 0, 0)


def _fused_body(idx_ref, tgt_ref, *refs):
    row_refs = refs[:K]
    emb_ref, loss_ref = refs[K], refs[K + 1]
    step = pl.program_id(0)

    @pl.when(step == 0)
    def _():
        loss_ref[0, 0] = 0.0

    _, sub, lan = row_refs[0].shape
    rows = jnp.concatenate([r[...] for r in row_refs], axis=0)
    emb_ref[...] = rows
    sub_iota = jax.lax.broadcasted_iota(jnp.int32, (K, sub, lan), 1)
    lane_iota = jax.lax.broadcasted_iota(jnp.int32, (K, sub, lan), 2)
    tgts = jnp.array([tgt_ref[step * K + k] for k in range(K)]).reshape(K, 1, 1)
    m = jnp.max(rows, axis=(1, 2), keepdims=True)
    s = jnp.sum(jnp.exp(rows - m), axis=(1, 2))
    lse = m.reshape(K) + jnp.log(s)
    hit = (sub_iota == tgts // lan) & (lane_iota == tgts % lan)
    tval = jnp.sum(jnp.where(hit, rows, 0.0), axis=(1, 2))
    loss_ref[0, 0] += jnp.sum(lse - tval)


def kernel(indices, targets, table):
    b, t = indices.shape
    n = b * t
    vocab, v = table.shape
    sub = v // LANES
    idx_flat = indices.reshape(n)
    tgt_flat = targets.reshape(n)
    table3 = table.reshape(vocab, sub, LANES)

    grid_spec = pltpu.PrefetchScalarGridSpec(
        num_scalar_prefetch=2,
        grid=(n // K,),
        in_specs=[
            pl.BlockSpec((1, sub, LANES), functools.partial(_row_imap, k=k))
            for k in range(K)
        ],
        out_specs=[
            pl.BlockSpec((K, sub, LANES), lambda i, idx_ref, tgt_ref: (i, 0, 0)),
            pl.BlockSpec(
                (1, 1),
                lambda i, idx_ref, tgt_ref: (0, 0),
                memory_space=pltpu.SMEM,
            ),
        ],
    )
    emb, loss_sum = pl.pallas_call(
        _fused_body,
        grid_spec=grid_spec,
        out_shape=[
            jax.ShapeDtypeStruct((n, sub, LANES), jnp.float32),
            jax.ShapeDtypeStruct((1, 1), jnp.float32),
        ],
    )(idx_flat, tgt_flat, *([table3] * K))
    embeddings = emb.reshape(b, t, v)
    loss = loss_sum[0, 0] / n
    return (embeddings, loss)


# speedup vs baseline: 4.4000x; 1.9110x over previous
import jax
import jax.numpy as jnp
from jax.experimental import pallas as pl
from jax.experimental.pallas import tpu as pltpu
from jax.experimental.pallas import tpu_sc as plsc

LANES = 128
GW = 4
R = 256


def _sc_gather(n, v, table, idx_padded):
    n_units = 32
    per_unit = n // n_units
    stride = 8

    @pl.kernel(
        out_type=jax.ShapeDtypeStruct((n, v), jnp.float32),
        mesh=plsc.VectorSubcoreMesh(core_axis_name="c", subcore_axis_name="s"),
        scratch_types=[
            pltpu.VMEM((per_unit // GW * stride,), jnp.int32),
            pltpu.VMEM((2, GW, v), jnp.float32),
            pltpu.SemaphoreType.DMA,
            pltpu.SemaphoreType.DMA,
            pltpu.SemaphoreType.DMA,
            pltpu.SemaphoreType.DMA,
        ],
    )
    def gather_kernel(table_hbm, i_hbm, o_hbm, idx_vmem, buf, g0, g1, s0, s1):
        core = jax.lax.axis_index("c")
        subcore = jax.lax.axis_index("s")
        u = core * 16 + subcore
        base = u * per_unit
        nwin = per_unit // GW

        pltpu.async_copy(
            i_hbm.at[0, pl.ds(u * (nwin * stride), nwin * stride)], idx_vmem, g0
        ).wait()

        def gather_dma(w, slot, sem):
            idx_win = idx_vmem.at[pl.ds(w * stride, GW)]
            return pltpu.make_async_copy(table_hbm.at[idx_win], buf.at[slot], sem)

        def write_dma(w, slot, sem):
            return pltpu.make_async_copy(
                buf.at[slot], o_hbm.at[pl.ds(base + w * GW, GW)], sem
            )

        gather_dma(0, 0, g0).start()

        @pl.loop(0, nwin, step=2)
        def _(w):
            gather_dma(w, 0, g0).wait()
            write_dma(w, 0, s0).start()

            @pl.when(w > 0)
            def _():
                write_dma(w - 1, 1, s1).wait()

            gather_dma(w + 1, 1, g1).start()
            gather_dma(w + 1, 1, g1).wait()
            write_dma(w + 1, 1, s1).start()

            @pl.when(w + 2 < nwin)
            def _():
                write_dma(w, 0, s0).wait()
                gather_dma(w + 2, 0, g0).start()

        write_dma(nwin - 2, 0, s0).wait()
        write_dma(nwin - 1, 1, s1).wait()

    return gather_kernel(table, idx_padded)


def _ce_body(emb_ref, tsub_ref, tlan_ref, loss_ref):
    step = pl.program_id(0)

    @pl.when(step == 0)
    def _():
        loss_ref[0, 0] = 0.0

    rows = emb_ref[...]
    r, sub, lan = rows.shape
    m = jnp.max(rows, axis=(1, 2), keepdims=True)
    s = jnp.sum(jnp.exp(rows - m), axis=(1, 2))
    lse = m.reshape(r) + jnp.log(s)
    tsub = tsub_ref[0].reshape(r, 1, 1)
    tlan = tlan_ref[0].reshape(r, 1, 1)
    sub_iota = jax.lax.broadcasted_iota(jnp.int32, (r, sub, lan), 1)
    lane_iota = jax.lax.broadcasted_iota(jnp.int32, (r, sub, lan), 2)
    hit = (sub_iota == tsub) & (lane_iota == tlan)
    tval = jnp.sum(jnp.where(hit, rows, 0.0), axis=(1, 2))
    loss_ref[0, 0] += jnp.sum(lse - tval)


def _tc_ce(n, v, emb3, tsub3, tlan3):
    sub = v // LANES
    loss_sum = pl.pallas_call(
        _ce_body,
        grid=(n // R,),
        in_specs=[
            pl.BlockSpec((R, sub, LANES), lambda i: (i, 0, 0)),
            pl.BlockSpec((1, 8, R // 8), lambda i: (i, 0, 0)),
            pl.BlockSpec((1, 8, R // 8), lambda i: (i, 0, 0)),
        ],
        out_specs=pl.BlockSpec((1, 1), lambda i: (0, 0), memory_space=pltpu.SMEM),
        out_shape=jax.ShapeDtypeStruct((1, 1), jnp.float32),
    )(emb3, tsub3, tlan3)
    return loss_sum[0, 0]


def kernel(indices, targets, table):
    b, t = indices.shape
    n = b * t
    vocab, v = table.shape
    sub = v // LANES
    tgt_flat = targets.reshape(n)
    tsub3 = (tgt_flat // LANES).reshape(n // R, 8, R // 8)
    tlan3 = (tgt_flat % LANES).reshape(n // R, 8, R // 8)
    idx_win4 = indices.reshape(n // GW, GW)
    idx_padded = jnp.pad(idx_win4, ((0, 0), (0, 8 - GW))).reshape(1, n // GW * 8)

    emb = _sc_gather(n, v, table, idx_padded)
    emb3 = emb.reshape(n, sub, LANES)
    loss_sum = _tc_ce(n, v, emb3, tsub3, tlan3)
    embeddings = emb.reshape(b, t, v)
    loss = loss_sum / n
    return (embeddings, loss)


# speedup vs baseline: 7.0781x; 1.6086x over previous
import jax
import jax.numpy as jnp
from jax.experimental import pallas as pl
from jax.experimental.pallas import tpu as pltpu
from jax.experimental.pallas import tpu_sc as plsc

LANES = 128
GW = 4
R = 256


def _sc_gather(n, v, table, idx_padded):
    n_units = 32
    per_unit = n // n_units
    stride = 8

    @pl.kernel(
        out_type=jax.ShapeDtypeStruct((n, v), jnp.float32),
        mesh=plsc.VectorSubcoreMesh(core_axis_name="c", subcore_axis_name="s"),
        scratch_types=[
            pltpu.VMEM((per_unit // GW * stride,), jnp.int32),
            pltpu.VMEM((2, GW, v), jnp.float32),
            pltpu.SemaphoreType.DMA,
            pltpu.SemaphoreType.DMA,
            pltpu.SemaphoreType.DMA,
            pltpu.SemaphoreType.DMA,
        ],
    )
    def gather_kernel(table_hbm, i_hbm, o_hbm, idx_vmem, buf, g0, g1, s0, s1):
        core = jax.lax.axis_index("c")
        subcore = jax.lax.axis_index("s")
        u = core * 16 + subcore
        base = u * per_unit
        nwin = per_unit // GW

        pltpu.async_copy(
            i_hbm.at[0, pl.ds(u * (nwin * stride), nwin * stride)], idx_vmem, g0
        ).wait()

        def gather_dma(w, slot, sem):
            idx_win = idx_vmem.at[pl.ds(w * stride, GW)]
            return pltpu.make_async_copy(table_hbm.at[idx_win], buf.at[slot], sem)

        def write_dma(w, slot, sem):
            return pltpu.make_async_copy(
                buf.at[slot], o_hbm.at[pl.ds(base + w * GW, GW)], sem
            )

        gather_dma(0, 0, g0).start()

        @pl.loop(0, nwin, step=2)
        def _(w):
            gather_dma(w, 0, g0).wait()
            write_dma(w, 0, s0).start()

            @pl.when(w > 0)
            def _():
                write_dma(w - 1, 1, s1).wait()

            gather_dma(w + 1, 1, g1).start()
            gather_dma(w + 1, 1, g1).wait()
            write_dma(w + 1, 1, s1).start()

            @pl.when(w + 2 < nwin)
            def _():
                write_dma(w, 0, s0).wait()
                gather_dma(w + 2, 0, g0).start()

        write_dma(nwin - 2, 0, s0).wait()
        write_dma(nwin - 1, 1, s1).wait()

    return gather_kernel(table, idx_padded)


def _ce_body(emb_ref, tgt_ref, loss_ref):
    step = pl.program_id(0)

    @pl.when(step == 0)
    def _():
        loss_ref[0, 0] = 0.0

    rows = emb_ref[...]
    r, v = rows.shape
    m = jnp.max(rows, axis=1, keepdims=True)
    s = jnp.sum(jnp.exp(rows - m), axis=1)
    lse = m.reshape(r) + jnp.log(s)
    tg = tgt_ref[...]
    col_iota = jax.lax.broadcasted_iota(jnp.int32, (r, v), 1)
    tval = jnp.sum(jnp.where(col_iota == tg, rows, 0.0), axis=1)
    loss_ref[0, 0] += jnp.sum(lse - tval)


def _tc_ce(n, v, emb, tgtcol):
    loss_sum = pl.pallas_call(
        _ce_body,
        grid=(n // R,),
        in_specs=[
            pl.BlockSpec((R, v), lambda i: (i, 0)),
            pl.BlockSpec((R, 1), lambda i: (i, 0)),
        ],
        out_specs=pl.BlockSpec((1, 1), lambda i: (0, 0), memory_space=pltpu.SMEM),
        out_shape=jax.ShapeDtypeStruct((1, 1), jnp.float32),
    )(emb, tgtcol)
    return loss_sum[0, 0]


def kernel(indices, targets, table):
    b, t = indices.shape
    n = b * t
    vocab, v = table.shape
    sub = v // LANES
    tgtcol = targets.reshape(n, 1)
    idx_win4 = indices.reshape(n // GW, GW)
    idx_padded = jnp.pad(idx_win4, ((0, 0), (0, 8 - GW))).reshape(1, n // GW * 8)

    emb = _sc_gather(n, v, table, idx_padded)
    loss_sum = _tc_ce(n, v, emb, tgtcol)
    embeddings = emb.reshape(b, t, v)
    loss = loss_sum / n
    return (embeddings, loss)
